# baseline (speedup 1.0000x reference)
import functools

import jax
import jax.numpy as jnp
from jax import lax
from jax.experimental import pallas as pl
from jax.experimental.pallas import tpu as pltpu
from jax.experimental.pallas import tpu_sc as plsc

N_TYPES = 3
EMB_D = 64
LANES = 16


@functools.lru_cache(maxsize=None)
def _make_lookup(nb: int, s: int):
    info = plsc.get_sparse_core_info()
    nw = info.num_cores * info.num_subcores
    b_per_w = nb // nw
    ngrp = b_per_w // LANES
    assert nb % (nw * LANES) == 0 and s % 2 == 0

    mesh = plsc.VectorSubcoreMesh(core_axis_name="c", subcore_axis_name="s")

    @functools.partial(
        pl.kernel,
        mesh=mesh,
        compiler_params=pltpu.CompilerParams(use_tc_tiling_on_sc=False),
        out_type=jax.ShapeDtypeStruct(
            (s, EMB_D // 8, nb // 128, 8, 128), jnp.float32),
        scratch_types=[
            pltpu.VMEM((N_TYPES, EMB_D, LANES), jnp.float32),
            pltpu.VMEM((b_per_w,), jnp.int32),
            pltpu.VMEM((b_per_w,), jnp.int32),
            pltpu.VMEM((EMB_D // 8, b_per_w // 128, 8, 128), jnp.float32),
            pltpu.VMEM((EMB_D // 8, b_per_w // 128, 8, 128), jnp.float32),
            pltpu.SemaphoreType.DMA,
            pltpu.SemaphoreType.DMA,
            pltpu.SemaphoreType.DMA,
            pltpu.SemaphoreType.DMA,
        ],
    )
    def lookup(wsplat_hbm, idxt_hbm, out_hbm, wsplat_v, idxA, idxB,
               rowsA, rowsB, isemA, isemB, osemA, osemB):
        wid = lax.axis_index("s") * info.num_cores + lax.axis_index("c")
        bbase = wid * b_per_w
        tcbase = wid * (b_per_w // 128)

        pltpu.sync_copy(wsplat_hbm, wsplat_v)

        def fetch_idx(p, idx_v, isem):
            pltpu.async_copy(idxt_hbm.at[p, pl.ds(bbase, b_per_w)], idx_v, isem)

        def wait_idx(p, idx_v, isem):
            pltpu.make_async_copy(
                idxt_hbm.at[p, pl.ds(bbase, b_per_w)], idx_v, isem).wait()

        def out_slice(p):
            return out_hbm.at[p, pl.ds(0, EMB_D // 8),
                              pl.ds(tcbase, b_per_w // 128)]

        def put(p, rows, osem):
            pltpu.async_copy(rows, out_slice(p), osem)

        def wait_put(p, rows, osem):
            pltpu.make_async_copy(rows, out_slice(p), osem).wait()

        def fill(idx_v, rows):
            for gb in range(ngrp // 8):
                ms = []
                for i in range(8):
                    v = idx_v[pl.ds(LANES * (8 * gb + i), LANES)]
                    ms.append((v == 0, v == 1))

                def dbody(d, carry, gb=gb, ms=ms):
                    w0 = wsplat_v[0, d]
                    w1 = wsplat_v[1, d]
                    w2 = wsplat_v[2, d]
                    tr = lax.shift_right_logical(d, 3)
                    dm = d & 7
                    for i in range(8):
                        g = 8 * gb + i
                        m0, m1 = ms[i]
                        rows[tr, g // 8, dm, pl.ds(LANES * (g % 8), LANES)] = \
                            jnp.where(m0, w0, jnp.where(m1, w1, w2))
                    return carry

                lax.fori_loop(0, EMB_D, dbody, 0)

        fetch_idx(0, idxA, isemA)
        fetch_idx(1, idxB, isemB)

        def body(j, carry):
            p0 = 2 * j
            wait_idx(p0, idxA, isemA)

            @pl.when(p0 >= 2)
            def _():
                wait_put(p0 - 2, rowsA, osemA)

            fill(idxA, rowsA)
            put(p0, rowsA, osemA)

            @pl.when(p0 + 2 < s)
            def _():
                fetch_idx(p0 + 2, idxA, isemA)

            wait_idx(p0 + 1, idxB, isemB)

            @pl.when(p0 >= 2)
            def _():
                wait_put(p0 - 1, rowsB, osemB)

            fill(idxB, rowsB)
            put(p0 + 1, rowsB, osemB)

            @pl.when(p0 + 3 < s)
            def _():
                fetch_idx(p0 + 3, idxB, isemB)

            return carry

        lax.fori_loop(0, s // 2, body, 0)
        wait_put(s - 2, rowsA, osemA)
        wait_put(s - 1, rowsB, osemB)

    return lookup


def kernel(type_indices, embedding_weight):
    b, s = type_indices.shape
    idxt = type_indices.T.astype(jnp.int32)
    wsplat = jnp.broadcast_to(
        embedding_weight[:, :, None], (N_TYPES, EMB_D, LANES))
    out5 = _make_lookup(b, s)(wsplat, idxt)
    out_t = jnp.transpose(out5, (0, 1, 3, 2, 4)).reshape(s, EMB_D, b)
    return jnp.transpose(out_t, (2, 0, 1))
